# baseline (speedup 1.0000x reference)
import functools

import jax
import jax.numpy as jnp
from jax import lax
from jax.experimental import pallas as pl
from jax.experimental.pallas import tpu as pltpu
from jax.experimental.pallas import tpu_sc as plsc

N = 10000
E = 320000
D = 128
NP = 10240
EP = 327680
TRASH = 10016
NTILES = 32
EPT = EP // NTILES
KCH = 80
BCH = 128
NCHUNK = EP // BCH
KPT = NCHUNK // 16
RPS = NP // 16
G = 8
NG = NP // (G * 128)

_mesh = plsc.VectorSubcoreMesh(core_axis_name="c", subcore_axis_name="s")


@functools.partial(
    pl.kernel,
    mesh=_mesh,
    out_type=jax.ShapeDtypeStruct((NTILES, NP), jnp.float32),
    scratch_types=[
        pltpu.VMEM((EPT,), jnp.int32),
        pltpu.VMEM((NP,), jnp.float32),
    ],
    compiler_params=pltpu.CompilerParams(needs_layout_passes=False),
)
def _deg(dstb, degout, dstv, degv):
    cid = lax.axis_index("c")
    sid = lax.axis_index("s")
    tid = cid * 16 + sid
    pltpu.sync_copy(dstb.at[tid], dstv)
    ones = jnp.ones((16,), jnp.float32)
    zeros = jnp.zeros((16,), jnp.float32)

    def zbody(i, _):
        degv[pl.ds(i * 16, 16)] = zeros
        return 0

    lax.fori_loop(0, NP // 16, zbody, 0)

    def hbody(i, _):
        idx = dstv[pl.ds(i * 16, 16)]
        plsc.addupdate_scatter(degv, [idx], ones)
        return 0

    lax.fori_loop(0, EPT // 16, hbody, 0)
    pltpu.sync_copy(degv, degout.at[tid])


@functools.partial(
    pl.kernel,
    mesh=_mesh,
    out_type=jax.ShapeDtypeStruct((NP, D), jnp.float32),
    scratch_types=[
        pltpu.VMEM((2, BCH), jnp.int32),
        pltpu.VMEM((2, BCH), jnp.int32),
        pltpu.VMEM((BCH, D), jnp.float32),
        pltpu.VMEM((BCH, D), jnp.float32),
        pltpu.VMEM_SHARED((NP, D), jnp.float32),
        pltpu.SemaphoreType.DMA,
        pltpu.SemaphoreType.DMA,
        pltpu.SemaphoreType.DMA,
        pltpu.SemaphoreType.DMA,
    ],
    compiler_params=pltpu.CompilerParams(needs_layout_passes=False),
)
def _scat(hd, idxb, zrows, accp, ib0, ib1, rb0, rb1, accsh,
          isem0, isem1, gsem0, gsem1):
    cid = lax.axis_index("c")
    sid = lax.axis_index("s")

    @pl.when(cid == 0)
    def _():
        base = sid * KPT
        pltpu.sync_copy(zrows, accsh.at[pl.ds(sid * RPS, RPS)])
        pltpu.async_copy(idxb.at[base], ib0, isem0)
        pltpu.async_copy(idxb.at[base + 1], ib1, isem1)
        plsc.subcore_barrier()
        pltpu.make_async_copy(idxb.at[base], ib0, isem0).wait()
        pltpu.async_copy(hd.at[ib0.at[0]], rb0, gsem0)

        def body(j, _):
            c = base + 2 * j
            pltpu.make_async_copy(idxb.at[c + 1], ib1, isem1).wait()
            pltpu.async_copy(hd.at[ib1.at[0]], rb1, gsem1)
            pltpu.make_async_copy(hd.at[ib0.at[0]], rb0, gsem0).wait()
            pltpu.sync_copy(rb0, accsh.at[ib0.at[1]], add=True)

            @pl.when(j < KPT // 2 - 1)
            def _():
                pltpu.async_copy(idxb.at[c + 2], ib0, isem0)
                pltpu.make_async_copy(idxb.at[c + 2], ib0, isem0).wait()
                pltpu.async_copy(hd.at[ib0.at[0]], rb0, gsem0)

            pltpu.make_async_copy(hd.at[ib1.at[0]], rb1, gsem1).wait()
            pltpu.sync_copy(rb1, accsh.at[ib1.at[1]], add=True)

            @pl.when(j < KPT // 2 - 1)
            def _():
                pltpu.async_copy(idxb.at[c + 3], ib1, isem1)

            return 0

        lax.fori_loop(0, KPT // 2, body, 0)
        plsc.subcore_barrier()
        pltpu.sync_copy(accsh.at[pl.ds(sid * RPS, RPS)],
                        accp.at[pl.ds(sid * RPS, RPS)])


def _p0_body(degp_ref, x_ref, w_ref, dinv_ref, hd_ref):
    deg = jnp.sum(degp_ref[...], axis=0) + 1.0
    dinv = lax.rsqrt(deg)
    dinv_ref[...] = dinv
    h = lax.dot_general(x_ref[...], w_ref[...], (((2,), (0,)), ((), ())),
                        preferred_element_type=jnp.float32)
    hd_ref[...] = h * dinv[:, :, None]


def _p0(degp, x3, w0):
    return pl.pallas_call(
        _p0_body,
        grid=(NG,),
        in_specs=[
            pl.BlockSpec((NTILES, G, 128), lambda i: (0, i, 0)),
            pl.BlockSpec((G, 128, 128), lambda i: (i, 0, 0)),
            pl.BlockSpec((128, 128), lambda i: (0, 0)),
        ],
        out_specs=[
            pl.BlockSpec((G, 128), lambda i: (i, 0)),
            pl.BlockSpec((G, 128, 128), lambda i: (i, 0, 0)),
        ],
        out_shape=[
            jax.ShapeDtypeStruct((NP // 128, 128), jnp.float32),
            jax.ShapeDtypeStruct((NP // 128, 128, 128), jnp.float32),
        ],
    )(degp, x3, w0)


def _ln(acc_ref, hd_ref, dinv_ref, b_ref, g_ref, be_ref):
    dinv = dinv_ref[...]
    s = (acc_ref[...] + hd_ref[...]) * dinv[:, :, None] + b_ref[...][None]
    mu = jnp.mean(s, axis=-1, keepdims=True)
    var = jnp.mean((s - mu) ** 2, axis=-1, keepdims=True)
    return (s - mu) / jnp.sqrt(var + 1e-5) * g_ref[...][None] \
        + be_ref[...][None], dinv


def _post_mid_body(acc_ref, hd_ref, dinv_ref, b_ref, g_ref, be_ref, wn_ref,
                   hdn_ref):
    z, dinv = _ln(acc_ref, hd_ref, dinv_ref, b_ref, g_ref, be_ref)
    y = jnp.maximum(z, 0.0)
    h = lax.dot_general(y, wn_ref[...], (((2,), (0,)), ((), ())),
                        preferred_element_type=jnp.float32)
    hdn_ref[...] = h * dinv[:, :, None]


def _post_last_body(acc_ref, hd_ref, dinv_ref, b_ref, g_ref, be_ref, y_ref):
    z, _ = _ln(acc_ref, hd_ref, dinv_ref, b_ref, g_ref, be_ref)
    y_ref[...] = z


_SPEC_H3 = pl.BlockSpec((G, 128, 128), lambda i: (i, 0, 0))
_SPEC_ACC = _SPEC_H3
_SPEC_DI = pl.BlockSpec((G, 128), lambda i: (i, 0))
_SPEC_VEC = pl.BlockSpec((1, 128), lambda i: (0, 0))
_SPEC_W = pl.BlockSpec((128, 128), lambda i: (0, 0))
_H3_SHAPE = jax.ShapeDtypeStruct((NP // 128, 128, 128), jnp.float32)


def _post_mid(acc4, hd3, dinv2, b, g, be, wn):
    return pl.pallas_call(
        _post_mid_body,
        grid=(NG,),
        in_specs=[_SPEC_ACC, _SPEC_H3, _SPEC_DI, _SPEC_VEC, _SPEC_VEC,
                  _SPEC_VEC, _SPEC_W],
        out_specs=_SPEC_H3,
        out_shape=_H3_SHAPE,
    )(acc4, hd3, dinv2, b, g, be, wn)


def _post_last(acc4, hd3, dinv2, b, g, be):
    return pl.pallas_call(
        _post_last_body,
        grid=(NG,),
        in_specs=[_SPEC_ACC, _SPEC_H3, _SPEC_DI, _SPEC_VEC, _SPEC_VEC,
                  _SPEC_VEC],
        out_specs=_SPEC_H3,
        out_shape=_H3_SHAPE,
    )(acc4, hd3, dinv2, b, g, be)


def kernel(x, edge_index, W0, b0, W1, b1, W2, b2, g0, be0, g1, be1, g2, be2):
    src = edge_index[0].astype(jnp.int32)
    dst = edge_index[1].astype(jnp.int32)
    pad = EP - E
    fill = jnp.full((pad,), TRASH, jnp.int32)
    srcb = jnp.concatenate([src, fill]).reshape(NCHUNK, 1, BCH)
    dstb = jnp.concatenate([dst, fill]).reshape(NCHUNK, 1, BCH)
    idxb = jnp.concatenate([srcb, dstb], axis=1)
    x3 = jnp.pad(x, ((0, NP - N), (0, 0))).reshape(NP // 128, 128, 128)
    zrows = jnp.zeros((RPS, D), jnp.float32)

    degp = _deg(dstb.reshape(NTILES, EPT))
    dinv2, hd = _p0(degp.reshape(NTILES, NP // 128, 128), x3, W0)

    b0r, g0r, be0r = b0.reshape(1, D), g0.reshape(1, D), be0.reshape(1, D)
    b1r, g1r, be1r = b1.reshape(1, D), g1.reshape(1, D), be1.reshape(1, D)
    b2r, g2r, be2r = b2.reshape(1, D), g2.reshape(1, D), be2.reshape(1, D)

    acc = _scat(hd.reshape(NP, D), idxb, zrows)
    acc4 = acc.reshape(NP // 128, 128, 128)
    hd = _post_mid(acc4, hd, dinv2, b0r, g0r, be0r, W1)

    acc = _scat(hd.reshape(NP, D), idxb, zrows)
    acc4 = acc.reshape(NP // 128, 128, 128)
    hd = _post_mid(acc4, hd, dinv2, b1r, g1r, be1r, W2)

    acc = _scat(hd.reshape(NP, D), idxb, zrows)
    acc4 = acc.reshape(NP // 128, 128, 128)
    y = _post_last(acc4, hd, dinv2, b2r, g2r, be2r)

    return y.reshape(NP, D)[:N]


# speedup vs baseline: 1.4142x; 1.4142x over previous
import functools

import jax
import jax.numpy as jnp
from jax import lax
from jax.experimental import pallas as pl
from jax.experimental.pallas import tpu as pltpu
from jax.experimental.pallas import tpu_sc as plsc

N = 10000
E = 320000
D = 128
NP = 10240
EP = 327680
TRASH = 10016
NTILES = 32
EPT = EP // NTILES
KCH = 80
BCH = 128
NCHUNK = EP // BCH
KC0 = 152
KC1 = 2 * KCH - KC0
RPS = NP // 16
G = 8
NG = NP // (G * 128)

_mesh = plsc.VectorSubcoreMesh(core_axis_name="c", subcore_axis_name="s")


@functools.partial(
    pl.kernel,
    mesh=_mesh,
    out_type=jax.ShapeDtypeStruct((NTILES, NP), jnp.float32),
    scratch_types=[
        pltpu.VMEM((EPT,), jnp.int32),
        pltpu.VMEM((NP,), jnp.float32),
    ],
    compiler_params=pltpu.CompilerParams(needs_layout_passes=False),
)
def _deg(dstb, degout, dstv, degv):
    cid = lax.axis_index("c")
    sid = lax.axis_index("s")
    tid = cid * 16 + sid
    pltpu.sync_copy(dstb.at[tid], dstv)
    ones = jnp.ones((16,), jnp.float32)
    zeros = jnp.zeros((16,), jnp.float32)

    def zbody(i, _):
        degv[pl.ds(i * 16, 16)] = zeros
        return 0

    lax.fori_loop(0, NP // 16, zbody, 0)

    def hbody(i, _):
        idx = dstv[pl.ds(i * 16, 16)]
        plsc.addupdate_scatter(degv, [idx], ones)
        return 0

    lax.fori_loop(0, EPT // 16, hbody, 0)
    pltpu.sync_copy(degv, degout.at[tid])


@functools.partial(
    pl.kernel,
    mesh=_mesh,
    out_type=jax.ShapeDtypeStruct((2, NP, D), jnp.float32),
    scratch_types=[
        pltpu.VMEM((2, BCH), jnp.int32),
        pltpu.VMEM((2, BCH), jnp.int32),
        pltpu.VMEM((BCH, D), jnp.float32),
        pltpu.VMEM((BCH, D), jnp.float32),
        pltpu.VMEM_SHARED((NP, D), jnp.float32),
        pltpu.SemaphoreType.DMA,
        pltpu.SemaphoreType.DMA,
        pltpu.SemaphoreType.DMA,
        pltpu.SemaphoreType.DMA,
    ],
    compiler_params=pltpu.CompilerParams(needs_layout_passes=False),
)
def _scat(hd, idxb, zrows, accp, ib0, ib1, rb0, rb1, accsh,
          isem0, isem1, gsem0, gsem1):
    cid = lax.axis_index("c")
    sid = lax.axis_index("s")
    cnt = jnp.where(cid == 0, KC0, KC1)
    base = jnp.where(cid == 0, sid * KC0, 16 * KC0 + sid * KC1)
    half = cnt // 2
    pltpu.sync_copy(zrows, accsh.at[pl.ds(sid * RPS, RPS)])
    pltpu.async_copy(idxb.at[base], ib0, isem0)
    pltpu.async_copy(idxb.at[base + 1], ib1, isem1)
    plsc.subcore_barrier()
    pltpu.make_async_copy(idxb.at[base], ib0, isem0).wait()
    pltpu.async_copy(hd.at[ib0.at[0]], rb0, gsem0)

    def body(j, _):
        c = base + 2 * j
        pltpu.make_async_copy(idxb.at[c + 1], ib1, isem1).wait()
        pltpu.async_copy(hd.at[ib1.at[0]], rb1, gsem1)
        pltpu.make_async_copy(hd.at[ib0.at[0]], rb0, gsem0).wait()
        pltpu.sync_copy(rb0, accsh.at[ib0.at[1]], add=True)

        @pl.when(j < half - 1)
        def _():
            pltpu.async_copy(idxb.at[c + 2], ib0, isem0)
            pltpu.make_async_copy(idxb.at[c + 2], ib0, isem0).wait()
            pltpu.async_copy(hd.at[ib0.at[0]], rb0, gsem0)

        pltpu.make_async_copy(hd.at[ib1.at[0]], rb1, gsem1).wait()
        pltpu.sync_copy(rb1, accsh.at[ib1.at[1]], add=True)

        @pl.when(j < half - 1)
        def _():
            pltpu.async_copy(idxb.at[c + 3], ib1, isem1)

        return 0

    lax.fori_loop(0, half, body, 0)
    plsc.subcore_barrier()
    pltpu.sync_copy(accsh.at[pl.ds(sid * RPS, RPS)],
                    accp.at[cid, pl.ds(sid * RPS, RPS)])


def _p0_body(degp_ref, x_ref, w_ref, dinv_ref, hd_ref):
    deg = jnp.sum(degp_ref[...], axis=0) + 1.0
    dinv = lax.rsqrt(deg)
    dinv_ref[...] = dinv
    h = lax.dot_general(x_ref[...], w_ref[...], (((2,), (0,)), ((), ())),
                        preferred_element_type=jnp.float32)
    hd_ref[...] = h * dinv[:, :, None]


def _p0(degp, x3, w0):
    return pl.pallas_call(
        _p0_body,
        grid=(NG,),
        in_specs=[
            pl.BlockSpec((NTILES, G, 128), lambda i: (0, i, 0)),
            pl.BlockSpec((G, 128, 128), lambda i: (i, 0, 0)),
            pl.BlockSpec((128, 128), lambda i: (0, 0)),
        ],
        out_specs=[
            pl.BlockSpec((G, 128), lambda i: (i, 0)),
            pl.BlockSpec((G, 128, 128), lambda i: (i, 0, 0)),
        ],
        out_shape=[
            jax.ShapeDtypeStruct((NP // 128, 128), jnp.float32),
            jax.ShapeDtypeStruct((NP // 128, 128, 128), jnp.float32),
        ],
    )(degp, x3, w0)


def _ln(acc_ref, hd_ref, dinv_ref, b_ref, g_ref, be_ref):
    dinv = dinv_ref[...]
    s = (acc_ref[0] + acc_ref[1] + hd_ref[...]) * dinv[:, :, None] \
        + b_ref[...][None]
    mu = jnp.mean(s, axis=-1, keepdims=True)
    var = jnp.mean((s - mu) ** 2, axis=-1, keepdims=True)
    return (s - mu) / jnp.sqrt(var + 1e-5) * g_ref[...][None] \
        + be_ref[...][None], dinv


def _post_mid_body(acc_ref, hd_ref, dinv_ref, b_ref, g_ref, be_ref, wn_ref,
                   hdn_ref):
    z, dinv = _ln(acc_ref, hd_ref, dinv_ref, b_ref, g_ref, be_ref)
    y = jnp.maximum(z, 0.0)
    h = lax.dot_general(y, wn_ref[...], (((2,), (0,)), ((), ())),
                        preferred_element_type=jnp.float32)
    hdn_ref[...] = h * dinv[:, :, None]


def _post_last_body(acc_ref, hd_ref, dinv_ref, b_ref, g_ref, be_ref, y_ref):
    z, _ = _ln(acc_ref, hd_ref, dinv_ref, b_ref, g_ref, be_ref)
    y_ref[...] = z


_SPEC_ACC = pl.BlockSpec((2, G, 128, 128), lambda i: (0, i, 0, 0))
_SPEC_H3 = pl.BlockSpec((G, 128, 128), lambda i: (i, 0, 0))
_SPEC_DI = pl.BlockSpec((G, 128), lambda i: (i, 0))
_SPEC_VEC = pl.BlockSpec((1, 128), lambda i: (0, 0))
_SPEC_W = pl.BlockSpec((128, 128), lambda i: (0, 0))
_H3_SHAPE = jax.ShapeDtypeStruct((NP // 128, 128, 128), jnp.float32)


def _post_mid(acc4, hd3, dinv2, b, g, be, wn):
    return pl.pallas_call(
        _post_mid_body,
        grid=(NG,),
        in_specs=[_SPEC_ACC, _SPEC_H3, _SPEC_DI, _SPEC_VEC, _SPEC_VEC,
                  _SPEC_VEC, _SPEC_W],
        out_specs=_SPEC_H3,
        out_shape=_H3_SHAPE,
    )(acc4, hd3, dinv2, b, g, be, wn)


def _post_last(acc4, hd3, dinv2, b, g, be):
    return pl.pallas_call(
        _post_last_body,
        grid=(NG,),
        in_specs=[_SPEC_ACC, _SPEC_H3, _SPEC_DI, _SPEC_VEC, _SPEC_VEC,
                  _SPEC_VEC],
        out_specs=_SPEC_H3,
        out_shape=_H3_SHAPE,
    )(acc4, hd3, dinv2, b, g, be)


def kernel(x, edge_index, W0, b0, W1, b1, W2, b2, g0, be0, g1, be1, g2, be2):
    src = edge_index[0].astype(jnp.int32)
    dst = edge_index[1].astype(jnp.int32)
    pad = EP - E
    fill = jnp.full((pad,), TRASH, jnp.int32)
    srcb = jnp.concatenate([src, fill]).reshape(NCHUNK, 1, BCH)
    dstb = jnp.concatenate([dst, fill]).reshape(NCHUNK, 1, BCH)
    idxb = jnp.concatenate([srcb, dstb], axis=1)
    x3 = jnp.pad(x, ((0, NP - N), (0, 0))).reshape(NP // 128, 128, 128)
    zrows = jnp.zeros((RPS, D), jnp.float32)

    degp = _deg(dstb.reshape(NTILES, EPT))
    dinv2, hd = _p0(degp.reshape(NTILES, NP // 128, 128), x3, W0)

    b0r, g0r, be0r = b0.reshape(1, D), g0.reshape(1, D), be0.reshape(1, D)
    b1r, g1r, be1r = b1.reshape(1, D), g1.reshape(1, D), be1.reshape(1, D)
    b2r, g2r, be2r = b2.reshape(1, D), g2.reshape(1, D), be2.reshape(1, D)

    acc = _scat(hd.reshape(NP, D), idxb, zrows)
    acc4 = acc.reshape(2, NP // 128, 128, 128)
    hd = _post_mid(acc4, hd, dinv2, b0r, g0r, be0r, W1)

    acc = _scat(hd.reshape(NP, D), idxb, zrows)
    acc4 = acc.reshape(2, NP // 128, 128, 128)
    hd = _post_mid(acc4, hd, dinv2, b1r, g1r, be1r, W2)

    acc = _scat(hd.reshape(NP, D), idxb, zrows)
    acc4 = acc.reshape(2, NP // 128, 128, 128)
    y = _post_last(acc4, hd, dinv2, b2r, g2r, be2r)

    return y.reshape(NP, D)[:N]
